# Initial kernel scaffold; baseline (speedup 1.0000x reference)
import jax
import jax.numpy as jnp
from jax.experimental import pallas as pl


def kernel(x, target):
    raise NotImplementedError("write your pallas kernel here")


# speedup vs baseline: 2.5726x; 2.5726x over previous
import functools
import math

import jax
import jax.numpy as jnp
import numpy as np
from jax import lax
from jax.experimental import pallas as pl
from jax.experimental.pallas import tpu as pltpu
from jax.experimental.pallas import tpu_sc as plsc

N_ROWS = 8192
N_COLS = 32000
PAD = 0
EPS = float(np.float32(0.1 / (N_COLS - 1)))
CONF = 0.9
C_ROW = (N_COLS - 1) * EPS * math.log(EPS) + CONF * math.log(CONF)
CME = CONF - EPS

NC = 2
NS = 16
L = 16
NW = NC * NS
PER_W = N_ROWS // NW
CHUNK = 128
NCH = PER_W // CHUNK

@functools.cache
def _make_sc_gather():
    mesh = plsc.VectorSubcoreMesh(core_axis_name="c", subcore_axis_name="s")

    @functools.partial(
        pl.kernel,
        mesh=mesh,
        out_type=jax.ShapeDtypeStruct((NW, L), jnp.float32),
        scratch_types=[
            pltpu.VMEM((PER_W,), jnp.int32),
            pltpu.VMEM((NCH, CHUNK), jnp.int32),
            pltpu.VMEM((NCH, CHUNK), jnp.float32),
            pltpu.VMEM((L,), jnp.float32),
            pltpu.SemaphoreType.DMA,
        ],
    )
    def _sc_gather(xf_hbm, tgt_hbm, out_hbm, tgt_v, idx_v, val_v, acc_v, sem):
        wid = lax.axis_index("s") * NC + lax.axis_index("c")
        base = wid * PER_W
        pltpu.sync_copy(tgt_hbm.at[pl.ds(base, PER_W)], tgt_v)
        lane = lax.iota(jnp.int32, L)
        for c in range(NCH):
            for j in range(CHUNK // L):
                off = c * CHUNK + j * L
                t = tgt_v[pl.ds(off, L)]
                rows = base + off + lane
                idx_v[c, pl.ds(j * L, L)] = rows * N_COLS + t
        copies = [
            pltpu.async_copy(xf_hbm.at[idx_v.at[c]], val_v.at[c], sem)
            for c in range(NCH)
        ]
        for cp in copies:
            cp.wait()
        acc = jnp.zeros((L,), jnp.float32)
        for c in range(NCH):
            for j in range(CHUNK // L):
                t = tgt_v[pl.ds(c * CHUNK + j * L, L)]
                v = val_v[c, pl.ds(j * L, L)]
                acc = acc + jnp.where(t != PAD, v, 0.0)
        acc_v[...] = acc
        pltpu.sync_copy(acc_v, out_hbm.at[wid])

    return _sc_gather


ROW_BLK = 64
GRID = N_ROWS // ROW_BLK


def _tc_body(x_ref, w_ref, g_ref, out_ref):
    i = pl.program_id(0)

    @pl.when(i == 0)
    def _init():
        out_ref[0, 0] = 0.0

    xb = x_ref[...]
    wb = w_ref[...]
    rs = jnp.sum(xb, axis=1, keepdims=True)
    out_ref[0, 0] += jnp.sum(wb * (C_ROW - EPS * rs))

    @pl.when(i == GRID - 1)
    def _fold():
        out_ref[0, 0] += -CME * jnp.sum(g_ref[...])


_tc_call = pl.pallas_call(
    _tc_body,
    grid=(GRID,),
    in_specs=[
        pl.BlockSpec((ROW_BLK, N_COLS), lambda i: (i, 0)),
        pl.BlockSpec((ROW_BLK, 1), lambda i: (i, 0)),
        pl.BlockSpec((NW, L), lambda i: (0, 0)),
    ],
    out_specs=pl.BlockSpec(
        (1, 1), lambda i: (0, 0), memory_space=pltpu.SMEM
    ),
    out_shape=jax.ShapeDtypeStruct((1, 1), jnp.float32),
)


def kernel(x, target):
    tgt = target.astype(jnp.int32)
    parts = _make_sc_gather()(x.reshape(-1), tgt)
    w = jnp.where(tgt != PAD, 1.0, 0.0).astype(jnp.float32)[:, None]
    out = _tc_call(x, w, parts)
    return out[0, 0]


# speedup vs baseline: 2.5783x; 1.0022x over previous
import functools
import math

import jax
import jax.numpy as jnp
import numpy as np
from jax import lax
from jax.experimental import pallas as pl
from jax.experimental.pallas import tpu as pltpu
from jax.experimental.pallas import tpu_sc as plsc

N_ROWS = 8192
N_COLS = 32000
PAD = 0
EPS = float(np.float32(0.1 / (N_COLS - 1)))
CONF = 0.9
C_ROW = (N_COLS - 1) * EPS * math.log(EPS) + CONF * math.log(CONF)
CME = CONF - EPS

NC = 2
NS = 16
L = 16
NW = NC * NS
PER_W = N_ROWS // NW
CHUNK = 128
NCH = PER_W // CHUNK

@functools.cache
def _make_sc_gather():
    mesh = plsc.VectorSubcoreMesh(core_axis_name="c", subcore_axis_name="s")

    @functools.partial(
        pl.kernel,
        mesh=mesh,
        out_type=jax.ShapeDtypeStruct((NW, L), jnp.float32),
        scratch_types=[
            pltpu.VMEM((PER_W,), jnp.int32),
            pltpu.VMEM((NCH, CHUNK), jnp.int32),
            pltpu.VMEM((NCH, CHUNK), jnp.float32),
            pltpu.VMEM((L,), jnp.float32),
            pltpu.SemaphoreType.DMA,
        ],
    )
    def _sc_gather(xf_hbm, tgt_hbm, out_hbm, tgt_v, idx_v, val_v, acc_v, sem):
        wid = lax.axis_index("s") * NC + lax.axis_index("c")
        base = wid * PER_W
        pltpu.sync_copy(tgt_hbm.at[pl.ds(base, PER_W)], tgt_v)
        lane = lax.iota(jnp.int32, L)
        for c in range(NCH):
            for j in range(CHUNK // L):
                off = c * CHUNK + j * L
                t = tgt_v[pl.ds(off, L)]
                rows = base + off + lane
                idx_v[c, pl.ds(j * L, L)] = rows * N_COLS + t
        copies = [
            pltpu.async_copy(xf_hbm.at[idx_v.at[c]], val_v.at[c], sem)
            for c in range(NCH)
        ]
        for cp in copies:
            cp.wait()
        acc = jnp.zeros((L,), jnp.float32)
        for c in range(NCH):
            for j in range(CHUNK // L):
                t = tgt_v[pl.ds(c * CHUNK + j * L, L)]
                v = val_v[c, pl.ds(j * L, L)]
                acc = acc + jnp.where(t != PAD, v, 0.0)
        acc_v[...] = acc
        pltpu.sync_copy(acc_v, out_hbm.at[wid])

    return _sc_gather


ROW_BLK = 128
GRID = N_ROWS // ROW_BLK


def _tc_body(x_ref, w_ref, g_ref, out_ref):
    i = pl.program_id(0)

    @pl.when(i == 0)
    def _init():
        out_ref[0, 0] = 0.0

    xb = x_ref[...]
    wb = w_ref[...]
    rs = jnp.sum(xb, axis=1, keepdims=True)
    out_ref[0, 0] += jnp.sum(wb * (C_ROW - EPS * rs))

    @pl.when(i == GRID - 1)
    def _fold():
        out_ref[0, 0] += -CME * jnp.sum(g_ref[...])


_tc_call = pl.pallas_call(
    _tc_body,
    grid=(GRID,),
    in_specs=[
        pl.BlockSpec((ROW_BLK, N_COLS), lambda i: (i, 0)),
        pl.BlockSpec((ROW_BLK, 1), lambda i: (i, 0)),
        pl.BlockSpec((NW, L), lambda i: (0, 0)),
    ],
    out_specs=pl.BlockSpec(
        (1, 1), lambda i: (0, 0), memory_space=pltpu.SMEM
    ),
    out_shape=jax.ShapeDtypeStruct((1, 1), jnp.float32),
)


def kernel(x, target):
    tgt = target.astype(jnp.int32)
    parts = _make_sc_gather()(x.reshape(-1), tgt)
    w = jnp.where(tgt != PAD, 1.0, 0.0).astype(jnp.float32)[:, None]
    out = _tc_call(x, w, parts)
    return out[0, 0]
